# Initial kernel scaffold; baseline (speedup 1.0000x reference)
import jax
import jax.numpy as jnp
from jax.experimental import pallas as pl


def kernel(points, normals, areas, features_point, w_point, queries, neighbors, lengths, s, inv_delta_w, inv_delta_f):
    raise NotImplementedError("write your pallas kernel here")


# speedup vs baseline: 41.9812x; 41.9812x over previous
import functools

import numpy as np
import jax
import jax.numpy as jnp
from jax import lax
from jax.experimental import pallas as pl
from jax.experimental.pallas import tpu as pltpu
from jax.experimental.pallas import tpu_sc as plsc

_NC, _NS, _LANES = 2, 16, 16
_NW = _NC * _NS


def _prep_body(p_ref, n_ref, a_ref, w_ref, out_ref):
    p = p_ref[...]
    nv = n_ref[...]
    a = a_ref[...]
    wp = w_ref[...]
    nn = nv / (jnp.sqrt(jnp.sum(nv * nv, axis=1, keepdims=True)) + 1e-10)
    x = 4.0 * wp
    wpos = 0.25 * (jnp.maximum(x, 0.0) + jnp.log1p(jnp.exp(-jnp.abs(x))))
    c7 = a * wpos * np.float32(1.0 / (4.0 * np.pi))
    zeros = jnp.zeros(p.shape[:1] + (8,), jnp.float32)
    out_ref[...] = jnp.concatenate([p, nn, a, c7, zeros], axis=1)


def _pack_geom(points, normals, areas, w_point):
    n = points.shape[0]
    bn = 2048
    n_pad = ((n + bn - 1) // bn) * bn
    pad = n_pad - n
    pp = jnp.pad(points, ((0, pad), (0, 0)))
    nn = jnp.pad(normals, ((0, pad), (0, 0)))
    aa = jnp.pad(areas[:, None], ((0, pad), (0, 0)))
    ww = jnp.pad(w_point, ((0, pad), (0, 0)))
    return pl.pallas_call(
        _prep_body,
        grid=(n_pad // bn,),
        in_specs=[
            pl.BlockSpec((bn, 3), lambda i: (i, 0)),
            pl.BlockSpec((bn, 3), lambda i: (i, 0)),
            pl.BlockSpec((bn, 1), lambda i: (i, 0)),
            pl.BlockSpec((bn, 1), lambda i: (i, 0)),
        ],
        out_specs=pl.BlockSpec((bn, 16), lambda i: (i, 0)),
        out_shape=jax.ShapeDtypeStruct((n_pad, 16), jnp.float32),
    )(pp, nn, aa, ww)


def _rsqrt(x):
    i = plsc.bitcast(x, jnp.int32)
    i = jnp.int32(0x5F3759DF) - lax.shift_right_logical(i, 1)
    y = plsc.bitcast(i, jnp.float32)
    for _ in range(3):
        y = y * (1.5 - 0.5 * x * y * y)
    return y


def _erf_pos(x):
    t = 1.0 / (1.0 + 0.3275911 * x)
    poly = t * (0.254829592 + t * (-0.284496736 + t * (1.421413741
           + t * (-1.453152027 + t * 1.061405429))))
    big = 1.0 - poly * jnp.exp(-x * x)
    x2 = x * x
    small = x * (1.1283791671 + x2 * (-0.3761263890 + x2 * (0.1128379167
            + x2 * (-0.0268661706 + x2 * (0.0052239776 + x2 * -0.0008548327)))))
    return jnp.where(x < 0.6, small, big)


def _full(v):
    return jnp.full((_LANES,), v, jnp.int32)


def _sc_body(qw, qb, k_nbr, nf,
             geom_hbm, feat_hbm, qry_hbm, neigh_hbm, len_hbm, consts_hbm,
             w_hbm, f_hbm, occ_hbm,
             qry_v, len_v, consts_v, nidx_v, geom_v, feat_v, kern_v, invden_v,
             wout_v, occout_v, fout_v, sem):
    wid = lax.axis_index("s") * _NC + lax.axis_index("c")
    q0 = pl.multiple_of(wid * qw, qw)
    idx_rows = qb * k_nbr // 128
    nblk = qw // qb
    nh = nf // _LANES

    pltpu.sync_copy(qry_hbm.at[pl.ds(q0, qw)], qry_v)
    pltpu.sync_copy(len_hbm.at[pl.ds(q0, qw)], len_v)
    pltpu.sync_copy(consts_hbm, consts_v)
    lane = lax.iota(jnp.int32, _LANES)
    zeros16 = jnp.zeros((_LANES,), jnp.float32)
    cwv = consts_v[0, :]
    cf2v = consts_v[1, :]
    esv = consts_v[2, :]

    def block_body(b, _):
        row0 = pl.multiple_of((q0 * k_nbr) // 128 + b * idx_rows, idx_rows)
        pltpu.sync_copy(neigh_hbm.at[pl.ds(row0, idx_rows)], nidx_v)
        copies = []
        for j in range(idx_rows):
            copies.append(pltpu.async_copy(
                geom_hbm.at[nidx_v.at[j]], geom_v.at[pl.ds(j * 128, 128)], sem))
            copies.append(pltpu.async_copy(
                feat_hbm.at[nidx_v.at[j]], feat_v.at[pl.ds(j * 128, 128)], sem))
        for c in copies:
            c.wait()

        for sub in range(qb // _LANES):
            qloc = b * qb + sub * _LANES
            qrow = jnp.full((_LANES,), qloc, jnp.int32) + lane
            qxv = plsc.load_gather(qry_v, [qrow, _full(0)])
            qyv = plsc.load_gather(qry_v, [qrow, _full(1)])
            qzv = plsc.load_gather(qry_v, [qrow, _full(2)])
            lnv = len_v[pl.ds(qloc, _LANES)]

            def j_body(j, carry):
                w_acc, k_acc = carry
                rows = (lane + sub * _LANES) * k_nbr + j
                jc = jnp.full((_LANES,), j, jnp.int32)
                col = lambda c: plsc.load_gather(geom_v, [rows, _full(c)])
                dx = col(0) - qxv
                dy = col(1) - qyv
                dz = col(2) - qzv
                proj = dx * col(3) + dy * col(4) + dz * col(5)
                d2 = dx * dx + dy * dy + dz * dz
                s2 = d2 + 1e-12
                dist = s2 * _rsqrt(s2)
                rr = _rsqrt(d2 + 1e-8)
                smooth = _erf_pos(dist * cwv)
                dens = col(7) * proj * (rr * rr * rr) * smooth
                kern = col(6) * jnp.exp(-0.5 * cf2v * s2)
                mask = jc < lnv
                dens = jnp.where(mask, dens, 0.0)
                kern = jnp.where(mask, kern, 0.0)
                kern_v[j, :] = kern
                return w_acc + dens, k_acc + kern

            w_acc, k_acc = lax.fori_loop(0, k_nbr, j_body, (zeros16, zeros16))
            invden_v[...] = 1.0 / (k_acc + 1e-8)
            wout_v[pl.ds(sub * _LANES, _LANES)] = w_acc
            occout_v[pl.ds(sub * _LANES, _LANES)] = (
                1.0 / (1.0 + jnp.exp(-esv * (w_acc - 0.5))))

            def q_body(ql, _):
                scale = plsc.load_gather(
                    invden_v, [jnp.full((_LANES,), ql, jnp.int32)])
                base = (sub * _LANES + ql) * k_nbr
                facc = [zeros16 for _ in range(nh)]
                for k in range(k_nbr):
                    bk = plsc.load_gather(
                        kern_v, [_full(k), jnp.full((_LANES,), ql, jnp.int32)])
                    for h in range(nh):
                        facc[h] = facc[h] + bk * feat_v[
                            base + k, pl.ds(h * _LANES, _LANES)]
                for h in range(nh):
                    fout_v[sub * _LANES + ql, pl.ds(h * _LANES, _LANES)] = (
                        facc[h] * scale)
                return 0

            lax.fori_loop(0, _LANES, q_body, 0)

        qg0 = pl.multiple_of(q0 + b * qb, qb)
        pltpu.sync_copy(wout_v, w_hbm.at[pl.ds(qg0, qb)])
        pltpu.sync_copy(occout_v, occ_hbm.at[pl.ds(qg0, qb)])
        pltpu.sync_copy(fout_v, f_hbm.at[pl.ds(qg0, qb)])
        return 0

    lax.fori_loop(0, nblk, block_body, 0)


def kernel(points, normals, areas, features_point, w_point, queries,
           neighbors, lengths, s, inv_delta_w, inv_delta_f):
    q, k_nbr = neighbors.shape
    nf = features_point.shape[1]
    qw = q // _NW
    qb = 32

    geom = _pack_geom(points, normals, areas, w_point)
    neigh2 = neighbors.reshape(q * k_nbr // 128, 128)
    consts = jnp.concatenate([
        (inv_delta_w * np.float32(1.0 / np.sqrt(2.0)))[None],
        (inv_delta_f * inv_delta_f)[None],
        jnp.exp(s)[None],
        jnp.zeros((5,), jnp.float32),
    ]).astype(jnp.float32)
    consts = jnp.broadcast_to(consts[:, None], (8, _LANES))

    mesh = plsc.VectorSubcoreMesh(core_axis_name="c", subcore_axis_name="s")
    body = functools.partial(_sc_body, qw, qb, k_nbr, nf)
    w, f, occ = pl.kernel(
        body,
        out_type=[
            jax.ShapeDtypeStruct((q,), jnp.float32),
            jax.ShapeDtypeStruct((q, nf), jnp.float32),
            jax.ShapeDtypeStruct((q,), jnp.float32),
        ],
        mesh=mesh,
        compiler_params=pltpu.CompilerParams(
            needs_layout_passes=False, use_tc_tiling_on_sc=False),
        scratch_types=[
            pltpu.VMEM((qw, 3), jnp.float32),
            pltpu.VMEM((qw,), jnp.int32),
            pltpu.VMEM((8, _LANES), jnp.float32),
            pltpu.VMEM((qb * k_nbr // 128, 128), jnp.int32),
            pltpu.VMEM((qb * k_nbr, 16), jnp.float32),
            pltpu.VMEM((qb * k_nbr, nf), jnp.float32),
            pltpu.VMEM((k_nbr, _LANES), jnp.float32),
            pltpu.VMEM((_LANES,), jnp.float32),
            pltpu.VMEM((qb,), jnp.float32),
            pltpu.VMEM((qb,), jnp.float32),
            pltpu.VMEM((qb, nf), jnp.float32),
            pltpu.SemaphoreType.DMA,
        ],
    )(geom, features_point, queries, neigh2, lengths, consts)
    return jnp.concatenate([w[:, None], f, occ[:, None]], axis=-1)


# speedup vs baseline: 47.9556x; 1.1423x over previous
import functools

import numpy as np
import jax
import jax.numpy as jnp
from jax import lax
from jax.experimental import pallas as pl
from jax.experimental.pallas import tpu as pltpu
from jax.experimental.pallas import tpu_sc as plsc

_NC, _NS, _LANES = 2, 16, 16
_NW = _NC * _NS


def _log1p_unit(u):
    t = u / (2.0 + u)
    t2 = t * t
    return 2.0 * t * (1.0 + t2 * (1.0 / 3.0 + t2 * (0.2 + t2 * (1.0 / 7.0
           + t2 * (1.0 / 9.0)))))


def _prep_sc_body(n_rows, pts_hbm, nrm_hbm, ar_hbm, wp_hbm, geom_hbm,
                  pts_v, nrm_v, ar_v, wp_v, geom_v):
    wid = lax.axis_index("s") * _NC + lax.axis_index("c")
    ch = 128
    nch_full = n_rows // ch
    tail = n_rows - nch_full * ch
    rounds = (nch_full + _NW - 1) // _NW
    lane = lax.iota(jnp.int32, _LANES)
    inv4pi = np.float32(1.0 / (4.0 * np.pi))

    def do_rows(nr):
        for g in range(nr // _LANES):
            rows = jnp.full((_LANES,), g * _LANES, jnp.int32) + lane
            px = plsc.load_gather(pts_v, [rows, _full(0)])
            py = plsc.load_gather(pts_v, [rows, _full(1)])
            pz = plsc.load_gather(pts_v, [rows, _full(2)])
            nx = plsc.load_gather(nrm_v, [rows, _full(0)])
            ny = plsc.load_gather(nrm_v, [rows, _full(1)])
            nz = plsc.load_gather(nrm_v, [rows, _full(2)])
            a = ar_v[pl.ds(g * _LANES, _LANES)]
            wp = plsc.load_gather(wp_v, [rows, _full(0)])
            nn = nx * nx + ny * ny + nz * nz
            inv_nrm = 1.0 / (nn * _rsqrt(nn) + 1e-10)
            x = 4.0 * wp
            u = jnp.exp(-jnp.abs(x))
            wpos = 0.25 * (jnp.maximum(x, 0.0) + _log1p_unit(u))
            vals = [px, py, pz, nx * inv_nrm, ny * inv_nrm, nz * inv_nrm,
                    a, a * wpos * inv4pi]
            for c, v in enumerate(vals):
                plsc.store_scatter(geom_v, [rows, _full(c)], v)

    def chunk_body(ci, _):
        c = ci * _NW + wid

        @pl.when(c < nch_full)
        def _():
            r0 = pl.multiple_of(c * ch, ch)
            pltpu.sync_copy(pts_hbm.at[pl.ds(r0, ch)], pts_v)
            pltpu.sync_copy(nrm_hbm.at[pl.ds(r0, ch)], nrm_v)
            pltpu.sync_copy(ar_hbm.at[pl.ds(r0, ch)], ar_v)
            pltpu.sync_copy(wp_hbm.at[pl.ds(r0, ch)], wp_v)
            do_rows(ch)
            pltpu.sync_copy(geom_v, geom_hbm.at[pl.ds(r0, ch)])

        return 0

    lax.fori_loop(0, rounds, chunk_body, 0)

    if tail:
        @pl.when(wid == 0)
        def _():
            r0 = nch_full * ch
            pltpu.sync_copy(pts_hbm.at[pl.ds(r0, tail)], pts_v.at[pl.ds(0, tail)])
            pltpu.sync_copy(nrm_hbm.at[pl.ds(r0, tail)], nrm_v.at[pl.ds(0, tail)])
            pltpu.sync_copy(ar_hbm.at[pl.ds(r0, tail)], ar_v.at[pl.ds(0, tail)])
            pltpu.sync_copy(wp_hbm.at[pl.ds(r0, tail)], wp_v.at[pl.ds(0, tail)])
            do_rows(tail)
            pltpu.sync_copy(geom_v.at[pl.ds(0, tail)],
                            geom_hbm.at[pl.ds(r0, tail)])


def _pack_geom(points, normals, areas, w_point):
    n = points.shape[0]
    assert n % _LANES == 0 and (n % 128) % _LANES == 0
    mesh = plsc.VectorSubcoreMesh(core_axis_name="c", subcore_axis_name="s")
    return pl.kernel(
        functools.partial(_prep_sc_body, n),
        out_type=jax.ShapeDtypeStruct((n, 16), jnp.float32),
        mesh=mesh,
        compiler_params=pltpu.CompilerParams(
            needs_layout_passes=False, use_tc_tiling_on_sc=False),
        scratch_types=[
            pltpu.VMEM((128, 3), jnp.float32),
            pltpu.VMEM((128, 3), jnp.float32),
            pltpu.VMEM((128,), jnp.float32),
            pltpu.VMEM((128, 1), jnp.float32),
            pltpu.VMEM((128, 16), jnp.float32),
        ],
    )(points, normals, areas, w_point)


def _rsqrt(x):
    i = plsc.bitcast(x, jnp.int32)
    i = jnp.int32(0x5F3759DF) - lax.shift_right_logical(i, 1)
    y = plsc.bitcast(i, jnp.float32)
    for _ in range(3):
        y = y * (1.5 - 0.5 * x * y * y)
    return y


def _erf_pos(x):
    t = 1.0 / (1.0 + 0.3275911 * x)
    poly = t * (0.254829592 + t * (-0.284496736 + t * (1.421413741
           + t * (-1.453152027 + t * 1.061405429))))
    big = 1.0 - poly * jnp.exp(-x * x)
    x2 = x * x
    small = x * (1.1283791671 + x2 * (-0.3761263890 + x2 * (0.1128379167
            + x2 * (-0.0268661706 + x2 * (0.0052239776 + x2 * -0.0008548327)))))
    return jnp.where(x < 0.6, small, big)


def _full(v):
    return jnp.full((_LANES,), v, jnp.int32)


def _sc_body(qw, qb, k_nbr, nf,
             geom_hbm, feat_hbm, qry_hbm, neigh_hbm, len_hbm, consts_hbm,
             w_hbm, f_hbm, occ_hbm,
             qry_v, len_v, consts_v, nidx_v, geom_v, feat_v, kern_v, invden_v,
             wout_v, occout_v, fout_v, sem):
    wid = lax.axis_index("s") * _NC + lax.axis_index("c")
    q0 = pl.multiple_of(wid * qw, qw)
    idx_rows = qb * k_nbr // 128
    nblk = qw // qb
    nh = nf // _LANES

    pltpu.sync_copy(qry_hbm.at[pl.ds(q0, qw)], qry_v)
    pltpu.sync_copy(len_hbm.at[pl.ds(q0, qw)], len_v)
    pltpu.sync_copy(consts_hbm, consts_v)
    lane = lax.iota(jnp.int32, _LANES)
    zeros16 = jnp.zeros((_LANES,), jnp.float32)
    cwv = consts_v[0, :]
    cf2v = consts_v[1, :]
    esv = consts_v[2, :]

    def block_body(b, _):
        row0 = pl.multiple_of((q0 * k_nbr) // 128 + b * idx_rows, idx_rows)
        pltpu.sync_copy(neigh_hbm.at[pl.ds(row0, idx_rows)], nidx_v)
        copies = []
        for j in range(idx_rows):
            copies.append(pltpu.async_copy(
                geom_hbm.at[nidx_v.at[j]], geom_v.at[pl.ds(j * 128, 128)], sem))
            copies.append(pltpu.async_copy(
                feat_hbm.at[nidx_v.at[j]], feat_v.at[pl.ds(j * 128, 128)], sem))
        for c in copies:
            c.wait()

        for sub in range(qb // _LANES):
            qloc = b * qb + sub * _LANES
            qrow = jnp.full((_LANES,), qloc, jnp.int32) + lane
            qxv = plsc.load_gather(qry_v, [qrow, _full(0)])
            qyv = plsc.load_gather(qry_v, [qrow, _full(1)])
            qzv = plsc.load_gather(qry_v, [qrow, _full(2)])
            lnv = len_v[pl.ds(qloc, _LANES)]

            def j_body(j, carry):
                w_acc, k_acc = carry
                rows = (lane + sub * _LANES) * k_nbr + j
                jc = jnp.full((_LANES,), j, jnp.int32)
                col = lambda c: plsc.load_gather(geom_v, [rows, _full(c)])
                dx = col(0) - qxv
                dy = col(1) - qyv
                dz = col(2) - qzv
                proj = dx * col(3) + dy * col(4) + dz * col(5)
                d2 = dx * dx + dy * dy + dz * dz
                s2 = d2 + 1e-12
                dist = s2 * _rsqrt(s2)
                rr = _rsqrt(d2 + 1e-8)
                smooth = _erf_pos(dist * cwv)
                dens = col(7) * proj * (rr * rr * rr) * smooth
                kern = col(6) * jnp.exp(-0.5 * cf2v * s2)
                mask = jc < lnv
                dens = jnp.where(mask, dens, 0.0)
                kern = jnp.where(mask, kern, 0.0)
                kern_v[j, :] = kern
                return w_acc + dens, k_acc + kern

            w_acc, k_acc = lax.fori_loop(0, k_nbr, j_body, (zeros16, zeros16))
            invden_v[...] = 1.0 / (k_acc + 1e-8)
            wout_v[pl.ds(sub * _LANES, _LANES)] = w_acc
            occout_v[pl.ds(sub * _LANES, _LANES)] = (
                1.0 / (1.0 + jnp.exp(-esv * (w_acc - 0.5))))

            def q_body(ql, _):
                scale = plsc.load_gather(
                    invden_v, [jnp.full((_LANES,), ql, jnp.int32)])
                base = (sub * _LANES + ql) * k_nbr
                facc = [zeros16 for _ in range(nh)]
                for k in range(k_nbr):
                    bk = plsc.load_gather(
                        kern_v, [_full(k), jnp.full((_LANES,), ql, jnp.int32)])
                    for h in range(nh):
                        facc[h] = facc[h] + bk * feat_v[
                            base + k, pl.ds(h * _LANES, _LANES)]
                for h in range(nh):
                    fout_v[sub * _LANES + ql, pl.ds(h * _LANES, _LANES)] = (
                        facc[h] * scale)
                return 0

            lax.fori_loop(0, _LANES, q_body, 0)

        qg0 = pl.multiple_of(q0 + b * qb, qb)
        pltpu.sync_copy(wout_v, w_hbm.at[pl.ds(qg0, qb)])
        pltpu.sync_copy(occout_v, occ_hbm.at[pl.ds(qg0, qb)])
        pltpu.sync_copy(fout_v, f_hbm.at[pl.ds(qg0, qb)])
        return 0

    lax.fori_loop(0, nblk, block_body, 0)


def kernel(points, normals, areas, features_point, w_point, queries,
           neighbors, lengths, s, inv_delta_w, inv_delta_f):
    q, k_nbr = neighbors.shape
    nf = features_point.shape[1]
    qw = q // _NW
    qb = 32

    geom = _pack_geom(points, normals, areas, w_point)
    neigh2 = neighbors.reshape(q * k_nbr // 128, 128)
    consts = jnp.concatenate([
        (inv_delta_w * np.float32(1.0 / np.sqrt(2.0)))[None],
        (inv_delta_f * inv_delta_f)[None],
        jnp.exp(s)[None],
        jnp.zeros((5,), jnp.float32),
    ]).astype(jnp.float32)
    consts = jnp.broadcast_to(consts[:, None], (8, _LANES))

    mesh = plsc.VectorSubcoreMesh(core_axis_name="c", subcore_axis_name="s")
    body = functools.partial(_sc_body, qw, qb, k_nbr, nf)
    w, f, occ = pl.kernel(
        body,
        out_type=[
            jax.ShapeDtypeStruct((q,), jnp.float32),
            jax.ShapeDtypeStruct((q, nf), jnp.float32),
            jax.ShapeDtypeStruct((q,), jnp.float32),
        ],
        mesh=mesh,
        compiler_params=pltpu.CompilerParams(
            needs_layout_passes=False, use_tc_tiling_on_sc=False),
        scratch_types=[
            pltpu.VMEM((qw, 3), jnp.float32),
            pltpu.VMEM((qw,), jnp.int32),
            pltpu.VMEM((8, _LANES), jnp.float32),
            pltpu.VMEM((qb * k_nbr // 128, 128), jnp.int32),
            pltpu.VMEM((qb * k_nbr, 16), jnp.float32),
            pltpu.VMEM((qb * k_nbr, nf), jnp.float32),
            pltpu.VMEM((k_nbr, _LANES), jnp.float32),
            pltpu.VMEM((_LANES,), jnp.float32),
            pltpu.VMEM((qb,), jnp.float32),
            pltpu.VMEM((qb,), jnp.float32),
            pltpu.VMEM((qb, nf), jnp.float32),
            pltpu.SemaphoreType.DMA,
        ],
    )(geom, features_point, queries, neigh2, lengths, consts)
    return jnp.concatenate([w[:, None], f, occ[:, None]], axis=-1)


# speedup vs baseline: 68.3270x; 1.4248x over previous
import functools

import numpy as np
import jax
import jax.numpy as jnp
from jax import lax
from jax.experimental import pallas as pl
from jax.experimental.pallas import tpu as pltpu
from jax.experimental.pallas import tpu_sc as plsc

_NC, _NS, _LANES = 2, 16, 16
_NW = _NC * _NS


def _log1p_unit(u):
    t = u / (2.0 + u)
    t2 = t * t
    return 2.0 * t * (1.0 + t2 * (1.0 / 3.0 + t2 * (0.2 + t2 * (1.0 / 7.0
           + t2 * (1.0 / 9.0)))))


def _prep_sc_body(n_rows, raw_hbm, geom_hbm, raw_v, geom_v):
    wid = lax.axis_index("s") * _NC + lax.axis_index("c")
    ch = 1024
    nch_full = n_rows // ch
    tail = n_rows - nch_full * ch
    rounds = (nch_full + _NW - 1) // _NW
    lane = lax.iota(jnp.int32, _LANES)
    inv4pi = np.float32(1.0 / (4.0 * np.pi))

    def do_rows(nr):
        for g in range(nr // _LANES):
            rows = jnp.full((_LANES,), g * _LANES, jnp.int32) + lane
            base = rows * 8
            col = lambda c: plsc.load_gather(raw_v, [base + c])
            px, py, pz = col(0), col(1), col(2)
            nx, ny, nz = col(3), col(4), col(5)
            a = col(6)
            wp = col(7)
            nn = nx * nx + ny * ny + nz * nz
            inv_nrm = 1.0 / (nn * _rsqrt(nn) + 1e-10)
            x = 4.0 * wp
            u = jnp.exp(-jnp.abs(x))
            wpos = 0.25 * (jnp.maximum(x, 0.0) + _log1p_unit(u))
            vals = [px, py, pz, nx * inv_nrm, ny * inv_nrm, nz * inv_nrm,
                    a, a * wpos * inv4pi]
            for c, v in enumerate(vals):
                plsc.store_scatter(geom_v, [rows, _full(c)], v)

    def chunk_body(ci, _):
        c = ci * _NW + wid

        @pl.when(c < nch_full)
        def _():
            r0 = pl.multiple_of(c * ch, ch)
            pltpu.sync_copy(raw_hbm.at[pl.ds(r0 * 8, ch * 8)], raw_v)
            do_rows(ch)
            pltpu.sync_copy(geom_v, geom_hbm.at[pl.ds(r0, ch)])

        return 0

    lax.fori_loop(0, rounds, chunk_body, 0)

    if tail:
        @pl.when(wid == 0)
        def _():
            r0 = nch_full * ch
            pltpu.sync_copy(raw_hbm.at[pl.ds(r0 * 8, tail * 8)],
                            raw_v.at[pl.ds(0, tail * 8)])
            do_rows(tail)
            pltpu.sync_copy(geom_v.at[pl.ds(0, tail)],
                            geom_hbm.at[pl.ds(r0, tail)])


def _pack_geom(points, normals, areas, w_point):
    n = points.shape[0]
    assert n % _LANES == 0 and (n % 1024) % _LANES == 0
    raw = jnp.concatenate(
        [points, normals, areas[:, None], w_point], axis=1).reshape(-1)
    mesh = plsc.VectorSubcoreMesh(core_axis_name="c", subcore_axis_name="s")
    return pl.kernel(
        functools.partial(_prep_sc_body, n),
        out_type=jax.ShapeDtypeStruct((n, 16), jnp.float32),
        mesh=mesh,
        compiler_params=pltpu.CompilerParams(
            needs_layout_passes=False, use_tc_tiling_on_sc=False),
        scratch_types=[
            pltpu.VMEM((1024 * 8,), jnp.float32),
            pltpu.VMEM((1024, 16), jnp.float32),
        ],
    )(raw)


def _rsqrt(x):
    i = plsc.bitcast(x, jnp.int32)
    i = jnp.int32(0x5F3759DF) - lax.shift_right_logical(i, 1)
    y = plsc.bitcast(i, jnp.float32)
    for _ in range(3):
        y = y * (1.5 - 0.5 * x * y * y)
    return y


def _erf_pos(x):
    t = 1.0 / (1.0 + 0.3275911 * x)
    poly = t * (0.254829592 + t * (-0.284496736 + t * (1.421413741
           + t * (-1.453152027 + t * 1.061405429))))
    big = 1.0 - poly * jnp.exp(-x * x)
    x2 = x * x
    small = x * (1.1283791671 + x2 * (-0.3761263890 + x2 * (0.1128379167
            + x2 * (-0.0268661706 + x2 * (0.0052239776 + x2 * -0.0008548327)))))
    return jnp.where(x < 0.6, small, big)


def _full(v):
    return jnp.full((_LANES,), v, jnp.int32)


def _sc_body(qw, qb, k_nbr, nf,
             geom_hbm, feat_hbm, qry_hbm, neigh_hbm, len_hbm, consts_hbm,
             w_hbm, f_hbm, occ_hbm,
             qry_v, len_v, consts_v, nidx_v, geom_v, feat_v, kern_v, invden_v,
             wout_v, occout_v, fout_v, sem):
    wid = lax.axis_index("s") * _NC + lax.axis_index("c")
    q0 = pl.multiple_of(wid * qw, qw)
    idx_rows = qb * k_nbr // 128
    nblk = qw // qb
    nh = nf // _LANES

    pltpu.sync_copy(qry_hbm.at[pl.ds(q0 * 3, qw * 3)], qry_v)
    pltpu.sync_copy(len_hbm.at[pl.ds(q0, qw)], len_v)
    pltpu.sync_copy(consts_hbm, consts_v)
    lane = lax.iota(jnp.int32, _LANES)
    zeros16 = jnp.zeros((_LANES,), jnp.float32)
    cwv = consts_v[0, :]
    cf2v = consts_v[1, :]
    esv = consts_v[2, :]

    def block_body(b, _):
        row0 = pl.multiple_of((q0 * k_nbr) // 128 + b * idx_rows, idx_rows)
        pltpu.sync_copy(neigh_hbm.at[pl.ds(row0, idx_rows)], nidx_v)
        copies = []
        for j in range(idx_rows):
            copies.append(pltpu.async_copy(
                geom_hbm.at[nidx_v.at[j]], geom_v.at[pl.ds(j * 128, 128)], sem))
            copies.append(pltpu.async_copy(
                feat_hbm.at[nidx_v.at[j]], feat_v.at[pl.ds(j * 128, 128)], sem))
        for c in copies:
            c.wait()

        for sub in range(qb // _LANES):
            qloc = b * qb + sub * _LANES
            qrow3 = (jnp.full((_LANES,), qloc, jnp.int32) + lane) * 3
            qxv = plsc.load_gather(qry_v, [qrow3])
            qyv = plsc.load_gather(qry_v, [qrow3 + 1])
            qzv = plsc.load_gather(qry_v, [qrow3 + 2])
            lnv = len_v[pl.ds(qloc, _LANES)]

            def j_body(j, carry):
                w_acc, k_acc = carry
                rows = (lane + sub * _LANES) * k_nbr + j
                jc = jnp.full((_LANES,), j, jnp.int32)
                col = lambda c: plsc.load_gather(geom_v, [rows, _full(c)])
                dx = col(0) - qxv
                dy = col(1) - qyv
                dz = col(2) - qzv
                proj = dx * col(3) + dy * col(4) + dz * col(5)
                d2 = dx * dx + dy * dy + dz * dz
                s2 = d2 + 1e-12
                dist = s2 * _rsqrt(s2)
                rr = _rsqrt(d2 + 1e-8)
                smooth = _erf_pos(dist * cwv)
                dens = col(7) * proj * (rr * rr * rr) * smooth
                kern = col(6) * jnp.exp(-0.5 * cf2v * s2)
                mask = jc < lnv
                dens = jnp.where(mask, dens, 0.0)
                kern = jnp.where(mask, kern, 0.0)
                kern_v[j, :] = kern
                return w_acc + dens, k_acc + kern

            w_acc, k_acc = lax.fori_loop(0, k_nbr, j_body, (zeros16, zeros16))
            invden_v[...] = 1.0 / (k_acc + 1e-8)
            wout_v[pl.ds(sub * _LANES, _LANES)] = w_acc
            occout_v[pl.ds(sub * _LANES, _LANES)] = (
                1.0 / (1.0 + jnp.exp(-esv * (w_acc - 0.5))))

            def q_body(ql, _):
                scale = plsc.load_gather(
                    invden_v, [jnp.full((_LANES,), ql, jnp.int32)])
                base = (sub * _LANES + ql) * k_nbr
                facc = [zeros16 for _ in range(nh)]
                for k in range(k_nbr):
                    bk = plsc.load_gather(
                        kern_v, [_full(k), jnp.full((_LANES,), ql, jnp.int32)])
                    for h in range(nh):
                        facc[h] = facc[h] + bk * feat_v[
                            base + k, pl.ds(h * _LANES, _LANES)]
                for h in range(nh):
                    fout_v[sub * _LANES + ql, pl.ds(h * _LANES, _LANES)] = (
                        facc[h] * scale)
                return 0

            lax.fori_loop(0, _LANES, q_body, 0)

        qg0 = pl.multiple_of(q0 + b * qb, qb)
        pltpu.sync_copy(wout_v, w_hbm.at[pl.ds(qg0, qb)])
        pltpu.sync_copy(occout_v, occ_hbm.at[pl.ds(qg0, qb)])
        pltpu.sync_copy(fout_v, f_hbm.at[pl.ds(qg0, qb)])
        return 0

    lax.fori_loop(0, nblk, block_body, 0)


def kernel(points, normals, areas, features_point, w_point, queries,
           neighbors, lengths, s, inv_delta_w, inv_delta_f):
    q, k_nbr = neighbors.shape
    nf = features_point.shape[1]
    qw = q // _NW
    qb = 32

    geom = _pack_geom(points, normals, areas, w_point)
    neigh2 = neighbors.reshape(q * k_nbr // 128, 128)
    consts = jnp.concatenate([
        (inv_delta_w * np.float32(1.0 / np.sqrt(2.0)))[None],
        (inv_delta_f * inv_delta_f)[None],
        jnp.exp(s)[None],
        jnp.zeros((5,), jnp.float32),
    ]).astype(jnp.float32)
    consts = jnp.broadcast_to(consts[:, None], (8, _LANES))

    mesh = plsc.VectorSubcoreMesh(core_axis_name="c", subcore_axis_name="s")
    body = functools.partial(_sc_body, qw, qb, k_nbr, nf)
    w, f, occ = pl.kernel(
        body,
        out_type=[
            jax.ShapeDtypeStruct((q,), jnp.float32),
            jax.ShapeDtypeStruct((q, nf), jnp.float32),
            jax.ShapeDtypeStruct((q,), jnp.float32),
        ],
        mesh=mesh,
        compiler_params=pltpu.CompilerParams(
            needs_layout_passes=False, use_tc_tiling_on_sc=False),
        scratch_types=[
            pltpu.VMEM((qw * 3,), jnp.float32),
            pltpu.VMEM((qw,), jnp.int32),
            pltpu.VMEM((8, _LANES), jnp.float32),
            pltpu.VMEM((qb * k_nbr // 128, 128), jnp.int32),
            pltpu.VMEM((qb * k_nbr, 16), jnp.float32),
            pltpu.VMEM((qb * k_nbr, nf), jnp.float32),
            pltpu.VMEM((k_nbr, _LANES), jnp.float32),
            pltpu.VMEM((_LANES,), jnp.float32),
            pltpu.VMEM((qb,), jnp.float32),
            pltpu.VMEM((qb,), jnp.float32),
            pltpu.VMEM((qb, nf), jnp.float32),
            pltpu.SemaphoreType.DMA,
        ],
    )(geom, features_point, queries.reshape(-1), neigh2, lengths, consts)
    return jnp.concatenate([w[:, None], f, occ[:, None]], axis=-1)


# speedup vs baseline: 81.3799x; 1.1910x over previous
import functools

import numpy as np
import jax
import jax.numpy as jnp
from jax import lax
from jax.experimental import pallas as pl
from jax.experimental.pallas import tpu as pltpu
from jax.experimental.pallas import tpu_sc as plsc

_NC, _NS, _LANES = 2, 16, 16
_NW = _NC * _NS


def _log1p_unit(u):
    t = u / (2.0 + u)
    t2 = t * t
    return 2.0 * t * (1.0 + t2 * (1.0 / 3.0 + t2 * (0.2 + t2 * (1.0 / 7.0
           + t2 * (1.0 / 9.0)))))


def _prep_sc_body(n_rows, raw_hbm, geom_hbm, raw_v, geom_v):
    wid = lax.axis_index("s") * _NC + lax.axis_index("c")
    ch = 1024
    nch_full = n_rows // ch
    tail = n_rows - nch_full * ch
    rounds = (nch_full + _NW - 1) // _NW
    lane = lax.iota(jnp.int32, _LANES)
    inv4pi = np.float32(1.0 / (4.0 * np.pi))

    def do_rows(nr):
        for g in range(nr // _LANES):
            rows = jnp.full((_LANES,), g * _LANES, jnp.int32) + lane
            base = rows * 8
            col = lambda c: plsc.load_gather(raw_v, [base + c])
            px, py, pz = col(0), col(1), col(2)
            nx, ny, nz = col(3), col(4), col(5)
            a = col(6)
            wp = col(7)
            nn = nx * nx + ny * ny + nz * nz
            inv_nrm = 1.0 / (nn * _rsqrt(nn) + 1e-10)
            x = 4.0 * wp
            u = jnp.exp(-jnp.abs(x))
            wpos = 0.25 * (jnp.maximum(x, 0.0) + _log1p_unit(u))
            vals = [px, py, pz, nx * inv_nrm, ny * inv_nrm, nz * inv_nrm,
                    a, a * wpos * inv4pi]
            for c, v in enumerate(vals):
                plsc.store_scatter(geom_v, [rows, _full(c)], v)

    def chunk_body(ci, _):
        c = ci * _NW + wid

        @pl.when(c < nch_full)
        def _():
            r0 = pl.multiple_of(c * ch, ch)
            pltpu.sync_copy(raw_hbm.at[pl.ds(r0 * 8, ch * 8)], raw_v)
            do_rows(ch)
            pltpu.sync_copy(geom_v, geom_hbm.at[pl.ds(r0, ch)])

        return 0

    lax.fori_loop(0, rounds, chunk_body, 0)

    if tail:
        @pl.when(wid == 0)
        def _():
            r0 = nch_full * ch
            pltpu.sync_copy(raw_hbm.at[pl.ds(r0 * 8, tail * 8)],
                            raw_v.at[pl.ds(0, tail * 8)])
            do_rows(tail)
            pltpu.sync_copy(geom_v.at[pl.ds(0, tail)],
                            geom_hbm.at[pl.ds(r0, tail)])


def _pack_geom(points, normals, areas, w_point):
    n = points.shape[0]
    assert n % _LANES == 0 and (n % 1024) % _LANES == 0
    raw = jnp.concatenate(
        [points, normals, areas[:, None], w_point], axis=1).reshape(-1)
    mesh = plsc.VectorSubcoreMesh(core_axis_name="c", subcore_axis_name="s")
    return pl.kernel(
        functools.partial(_prep_sc_body, n),
        out_type=jax.ShapeDtypeStruct((n, 16), jnp.float32),
        mesh=mesh,
        compiler_params=pltpu.CompilerParams(
            needs_layout_passes=False, use_tc_tiling_on_sc=False),
        scratch_types=[
            pltpu.VMEM((1024 * 8,), jnp.float32),
            pltpu.VMEM((1024, 16), jnp.float32),
        ],
    )(raw)


def _rsqrt(x):
    i = plsc.bitcast(x, jnp.int32)
    i = jnp.int32(0x5F3759DF) - lax.shift_right_logical(i, 1)
    y = plsc.bitcast(i, jnp.float32)
    for _ in range(3):
        y = y * (1.5 - 0.5 * x * y * y)
    return y


def _erf_pos(x):
    t = 1.0 / (1.0 + 0.3275911 * x)
    poly = t * (0.254829592 + t * (-0.284496736 + t * (1.421413741
           + t * (-1.453152027 + t * 1.061405429))))
    big = 1.0 - poly * jnp.exp(-x * x)
    x2 = x * x
    small = x * (1.1283791671 + x2 * (-0.3761263890 + x2 * (0.1128379167
            + x2 * (-0.0268661706 + x2 * (0.0052239776 + x2 * -0.0008548327)))))
    return jnp.where(x < 0.6, small, big)


def _full(v):
    return jnp.full((_LANES,), v, jnp.int32)


def _sc_body(qw, qb, k_nbr, nf,
             geom_hbm, feat_hbm, qry_hbm, neigh_hbm, len_hbm, consts_hbm,
             w_hbm, f_hbm, occ_hbm,
             qry_v, len_v, consts_v, nidx0, nidx1, geom0, geom1, feat0, feat1,
             kern_v, invden_v, wout_v, occout_v, fout_v,
             semg0, semf0, semg1, semf1):
    wid = lax.axis_index("s") * _NC + lax.axis_index("c")
    q0 = pl.multiple_of(wid * qw, qw)
    idx_rows = qb * k_nbr // 128
    nblk = qw // qb
    assert nblk % 2 == 0
    nh = nf // _LANES

    pltpu.sync_copy(qry_hbm.at[pl.ds(q0 * 3, qw * 3)], qry_v)
    pltpu.sync_copy(len_hbm.at[pl.ds(q0, qw)], len_v)
    pltpu.sync_copy(consts_hbm, consts_v)
    lane = lax.iota(jnp.int32, _LANES)
    zeros16 = jnp.zeros((_LANES,), jnp.float32)
    cwv = consts_v[0, :]
    cf2v = consts_v[1, :]
    esv = consts_v[2, :]

    def issue(b, nidx_v, geom_v, feat_v, semg, semf):
        row0 = pl.multiple_of((q0 * k_nbr) // 128 + b * idx_rows, idx_rows)
        pltpu.sync_copy(neigh_hbm.at[pl.ds(row0, idx_rows)], nidx_v)
        for j in range(idx_rows):
            pltpu.async_copy(
                geom_hbm.at[nidx_v.at[j]], geom_v.at[pl.ds(j * 128, 128)], semg)
            pltpu.async_copy(
                feat_hbm.at[nidx_v.at[j]], feat_v.at[pl.ds(j * 128, 128)], semf)

    def drain(geom_v, feat_v, semg, semf):
        pltpu.make_async_copy(
            geom_hbm.at[pl.ds(0, qb * k_nbr)], geom_v, semg).wait()
        pltpu.make_async_copy(
            feat_hbm.at[pl.ds(0, qb * k_nbr)], feat_v, semf).wait()

    def compute(b, geom_v, feat_v):
        for sub in range(qb // _LANES):
            qloc = b * qb + sub * _LANES
            qrow3 = (jnp.full((_LANES,), qloc, jnp.int32) + lane) * 3
            qxv = plsc.load_gather(qry_v, [qrow3])
            qyv = plsc.load_gather(qry_v, [qrow3 + 1])
            qzv = plsc.load_gather(qry_v, [qrow3 + 2])
            lnv = len_v[pl.ds(qloc, _LANES)]

            def j_body(j, carry):
                w_acc, k_acc = carry
                rows = (lane + sub * _LANES) * k_nbr + j
                jc = jnp.full((_LANES,), j, jnp.int32)
                col = lambda c: plsc.load_gather(geom_v, [rows, _full(c)])
                dx = col(0) - qxv
                dy = col(1) - qyv
                dz = col(2) - qzv
                proj = dx * col(3) + dy * col(4) + dz * col(5)
                d2 = dx * dx + dy * dy + dz * dz
                s2 = d2 + 1e-12
                dist = s2 * _rsqrt(s2)
                rr = _rsqrt(d2 + 1e-8)
                smooth = _erf_pos(dist * cwv)
                dens = col(7) * proj * (rr * rr * rr) * smooth
                kern = col(6) * jnp.exp(-0.5 * cf2v * s2)
                mask = jc < lnv
                dens = jnp.where(mask, dens, 0.0)
                kern = jnp.where(mask, kern, 0.0)
                kern_v[j, :] = kern
                return w_acc + dens, k_acc + kern

            w_acc, k_acc = lax.fori_loop(0, k_nbr, j_body, (zeros16, zeros16))
            invden_v[...] = 1.0 / (k_acc + 1e-8)
            wout_v[pl.ds(sub * _LANES, _LANES)] = w_acc
            occout_v[pl.ds(sub * _LANES, _LANES)] = (
                1.0 / (1.0 + jnp.exp(-esv * (w_acc - 0.5))))

            def q_body(ql, _):
                scale = plsc.load_gather(
                    invden_v, [jnp.full((_LANES,), ql, jnp.int32)])
                base = (sub * _LANES + ql) * k_nbr
                facc = [zeros16 for _ in range(nh)]
                for k in range(k_nbr):
                    bk = plsc.load_gather(
                        kern_v, [_full(k), jnp.full((_LANES,), ql, jnp.int32)])
                    for h in range(nh):
                        facc[h] = facc[h] + bk * feat_v[
                            base + k, pl.ds(h * _LANES, _LANES)]
                for h in range(nh):
                    fout_v[sub * _LANES + ql, pl.ds(h * _LANES, _LANES)] = (
                        facc[h] * scale)
                return 0

            lax.fori_loop(0, _LANES, q_body, 0)

        qg0 = pl.multiple_of(q0 + b * qb, qb)
        pltpu.sync_copy(wout_v, w_hbm.at[pl.ds(qg0, qb)])
        pltpu.sync_copy(occout_v, occ_hbm.at[pl.ds(qg0, qb)])
        pltpu.sync_copy(fout_v, f_hbm.at[pl.ds(qg0, qb)])

    issue(0, nidx0, geom0, feat0, semg0, semf0)

    def pair_body(b2, _):
        b = b2 * 2
        drain(geom0, feat0, semg0, semf0)
        issue(b + 1, nidx1, geom1, feat1, semg1, semf1)
        compute(b, geom0, feat0)
        drain(geom1, feat1, semg1, semf1)

        @pl.when(b + 2 < nblk)
        def _():
            issue(b + 2, nidx0, geom0, feat0, semg0, semf0)

        compute(b + 1, geom1, feat1)
        return 0

    lax.fori_loop(0, nblk // 2, pair_body, 0)


def kernel(points, normals, areas, features_point, w_point, queries,
           neighbors, lengths, s, inv_delta_w, inv_delta_f):
    q, k_nbr = neighbors.shape
    nf = features_point.shape[1]
    qw = q // _NW
    qb = 32

    geom = _pack_geom(points, normals, areas, w_point)
    neigh2 = neighbors.reshape(q * k_nbr // 128, 128)
    consts = jnp.concatenate([
        (inv_delta_w * np.float32(1.0 / np.sqrt(2.0)))[None],
        (inv_delta_f * inv_delta_f)[None],
        jnp.exp(s)[None],
        jnp.zeros((5,), jnp.float32),
    ]).astype(jnp.float32)
    consts = jnp.broadcast_to(consts[:, None], (8, _LANES))

    mesh = plsc.VectorSubcoreMesh(core_axis_name="c", subcore_axis_name="s")
    body = functools.partial(_sc_body, qw, qb, k_nbr, nf)
    w, f, occ = pl.kernel(
        body,
        out_type=[
            jax.ShapeDtypeStruct((q,), jnp.float32),
            jax.ShapeDtypeStruct((q, nf), jnp.float32),
            jax.ShapeDtypeStruct((q,), jnp.float32),
        ],
        mesh=mesh,
        compiler_params=pltpu.CompilerParams(
            needs_layout_passes=False, use_tc_tiling_on_sc=False),
        scratch_types=[
            pltpu.VMEM((qw * 3,), jnp.float32),
            pltpu.VMEM((qw,), jnp.int32),
            pltpu.VMEM((8, _LANES), jnp.float32),
            pltpu.VMEM((qb * k_nbr // 128, 128), jnp.int32),
            pltpu.VMEM((qb * k_nbr // 128, 128), jnp.int32),
            pltpu.VMEM((qb * k_nbr, 16), jnp.float32),
            pltpu.VMEM((qb * k_nbr, 16), jnp.float32),
            pltpu.VMEM((qb * k_nbr, nf), jnp.float32),
            pltpu.VMEM((qb * k_nbr, nf), jnp.float32),
            pltpu.VMEM((k_nbr, _LANES), jnp.float32),
            pltpu.VMEM((_LANES,), jnp.float32),
            pltpu.VMEM((qb,), jnp.float32),
            pltpu.VMEM((qb,), jnp.float32),
            pltpu.VMEM((qb, nf), jnp.float32),
            pltpu.SemaphoreType.DMA,
            pltpu.SemaphoreType.DMA,
            pltpu.SemaphoreType.DMA,
            pltpu.SemaphoreType.DMA,
        ],
    )(geom, features_point, queries.reshape(-1), neigh2, lengths, consts)
    return jnp.concatenate([w[:, None], f, occ[:, None]], axis=-1)
